# Initial kernel scaffold; baseline (speedup 1.0000x reference)
import functools

import jax
import jax.numpy as jnp
from jax import lax
from jax.experimental import pallas as pl
from jax.experimental.pallas import tpu as pltpu
from jax.experimental.pallas import tpu_sc as plsc

N = 10000
E = 320000
D = 128
H = 128
PAD = 16
W = D + PAD
NC = 2
NS = 16
NW = NC * NS
EPW = E // NW
K = 80
NB = EPW // K
RPT = N // NS
RCH = 125


def _sc_segment_sum(xp, src2d, dst2d, zrows):
    mesh = plsc.VectorSubcoreMesh(core_axis_name="c", subcore_axis_name="s")

    @functools.partial(
        pl.kernel,
        out_type=jax.ShapeDtypeStruct((NC, N, W), jnp.float32),
        mesh=mesh,
        scratch_types=[
            pltpu.VMEM_SHARED((N, W), jnp.float32),
            pltpu.VMEM((NB, K), jnp.int32),
            pltpu.VMEM((NB, K), jnp.int32),
            pltpu.VMEM((RCH, W), jnp.float32),
            pltpu.VMEM((K, W), jnp.float32),
            pltpu.SemaphoreType.DMA,
        ],
    )
    def k(xp_hbm, src_hbm, dst_hbm, z_hbm, out_hbm,
          accum, src_v, dst_v, stage, rows, sem):
        cid = lax.axis_index("c")
        sid = lax.axis_index("s")
        wid = sid * NC + cid

        pltpu.sync_copy(z_hbm, stage)
        row0 = sid * RPT
        for t in range(RPT // RCH):
            pltpu.sync_copy(stage, accum.at[pl.ds(row0 + t * RCH, RCH)])

        base = wid * NB
        pltpu.sync_copy(src_hbm.at[pl.ds(base, NB)], src_v)
        pltpu.sync_copy(dst_hbm.at[pl.ds(base, NB)], dst_v)

        plsc.subcore_barrier()

        def step(i, carry):
            pltpu.async_copy(xp_hbm.at[src_v.at[i]], rows, sem).wait()
            pltpu.sync_copy(rows, accum.at[dst_v.at[i]], add=True)
            return carry

        lax.fori_loop(0, NB, step, 0)

        plsc.subcore_barrier()

        for t in range(RPT // RCH):
            r = row0 + t * RCH
            pltpu.sync_copy(accum.at[pl.ds(r, RCH)], stage)
            pltpu.sync_copy(stage, out_hbm.at[cid, pl.ds(r, RCH)])

    return k(xp, src2d, dst2d, zrows)


def _tc_dense(acc, x, W_l, b_l, W_r):
    BN = 1000
    grid = (N // BN,)

    def body(acc_ref, x_ref, wl_ref, b_ref, wr_ref, o_ref):
        a = acc_ref[0] + acc_ref[1]
        s = a[:, :D]
        c = jnp.sum(a[:, D:], axis=1, keepdims=True)
        mean = s / jnp.maximum(c, 1.0)
        out = lax.dot_general(mean, wl_ref[...], (((1,), (1,)), ((), ())),
                              preferred_element_type=jnp.float32)
        out = out + lax.dot_general(x_ref[...], wr_ref[...],
                                    (((1,), (1,)), ((), ())),
                                    preferred_element_type=jnp.float32)
        o_ref[...] = jnp.maximum(out + b_ref[...], 0.0)

    return pl.pallas_call(
        body,
        grid=grid,
        in_specs=[
            pl.BlockSpec((NC, BN, W), lambda i: (0, i, 0)),
            pl.BlockSpec((BN, D), lambda i: (i, 0)),
            pl.BlockSpec((H, D), lambda i: (0, 0)),
            pl.BlockSpec((1, H), lambda i: (0, 0)),
            pl.BlockSpec((H, D), lambda i: (0, 0)),
        ],
        out_specs=pl.BlockSpec((BN, H), lambda i: (i, 0)),
        out_shape=jax.ShapeDtypeStruct((N, H), jnp.float32),
    )(acc, x, W_l, b_l, W_r)


def kernel(x, edge_weight, W_l, b_l, W_r, edge_index):
    del edge_weight
    pad = jnp.zeros((N, PAD), jnp.float32).at[:, 0].set(1.0)
    xp = jnp.concatenate([x, pad], axis=1)
    src2d = edge_index[0].reshape(E // K, K)
    dst2d = edge_index[1].reshape(E // K, K)
    zrows = jnp.zeros((RCH, W), jnp.float32)
    acc = _sc_segment_sum(xp, src2d, dst2d, zrows)
    return _tc_dense(acc, x, W_l, b_l.reshape(1, H), W_r)


# speedup vs baseline: 7.4428x; 7.4428x over previous
import functools

import jax
import jax.numpy as jnp
from jax import lax
from jax.experimental import pallas as pl
from jax.experimental.pallas import tpu as pltpu
from jax.experimental.pallas import tpu_sc as plsc

N = 10000
E = 320000
D = 128
H = 128
PAD = 16
W = D + PAD
NC = 2
NS = 16
NW = NC * NS
EPW = E // NW
K = 80
NB = EPW // K
NA = 10240
RPT = NA // NS
CB = 25
NCH = NB // CB


def _sc_segment_sum(xp, src2d, dst2d, zrows):
    mesh = plsc.VectorSubcoreMesh(core_axis_name="c", subcore_axis_name="s")

    @functools.partial(
        pl.kernel,
        out_type=jax.ShapeDtypeStruct((NC, NA, W), jnp.float32),
        mesh=mesh,
        scratch_types=[
            pltpu.VMEM_SHARED((NA, W), jnp.float32),
            pltpu.VMEM((CB, K), jnp.int32),
            pltpu.VMEM((CB, K), jnp.int32),
            pltpu.VMEM((K, W), jnp.float32),
            pltpu.SemaphoreType.DMA,
        ],
        compiler_params=pltpu.CompilerParams(use_tc_tiling_on_sc=False),
    )
    def k(xp_hbm, src_hbm, dst_hbm, z_hbm, out_hbm,
          accum, src_v, dst_v, rows, sem):
        cid = lax.axis_index("c")
        sid = lax.axis_index("s")
        wid = sid * NC + cid

        pltpu.sync_copy(z_hbm, rows)
        row0 = sid * RPT
        for t in range(RPT // K):
            pltpu.sync_copy(rows, accum.at[pl.ds(row0 + t * K, K)])

        plsc.subcore_barrier()

        for c in range(NCH):
            pltpu.sync_copy(src_hbm.at[wid, pl.ds(c * CB, CB)], src_v)
            pltpu.sync_copy(dst_hbm.at[wid, pl.ds(c * CB, CB)], dst_v)

            def step(i, carry):
                pltpu.async_copy(xp_hbm.at[src_v.at[i]], rows, sem).wait()
                pltpu.sync_copy(rows, accum.at[dst_v.at[i]], add=True)
                return carry

            lax.fori_loop(0, CB, step, 0)

        plsc.subcore_barrier()

        for t in range(RPT // K):
            r = row0 + t * K
            pltpu.sync_copy(accum.at[pl.ds(r, K)], rows)
            pltpu.sync_copy(rows, out_hbm.at[cid, pl.ds(r, K)])

    return k(xp, src2d, dst2d, zrows)


def _tc_dense(acc, x, W_l, b_l, W_r):
    BN = 1000
    grid = (N // BN,)

    def body(acc_ref, x_ref, wl_ref, b_ref, wr_ref, o_ref):
        a = acc_ref[0] + acc_ref[1]
        s = a[:, :D]
        c = jnp.sum(a[:, D:], axis=1, keepdims=True)
        mean = s / jnp.maximum(c, 1.0)
        out = lax.dot_general(mean, wl_ref[...], (((1,), (1,)), ((), ())),
                              preferred_element_type=jnp.float32)
        out = out + lax.dot_general(x_ref[...], wr_ref[...],
                                    (((1,), (1,)), ((), ())),
                                    preferred_element_type=jnp.float32)
        o_ref[...] = jnp.maximum(out + b_ref[...], 0.0)

    return pl.pallas_call(
        body,
        grid=grid,
        in_specs=[
            pl.BlockSpec((NC, BN, W), lambda i: (0, i, 0)),
            pl.BlockSpec((BN, D), lambda i: (i, 0)),
            pl.BlockSpec((H, D), lambda i: (0, 0)),
            pl.BlockSpec((1, H), lambda i: (0, 0)),
            pl.BlockSpec((H, D), lambda i: (0, 0)),
        ],
        out_specs=pl.BlockSpec((BN, H), lambda i: (i, 0)),
        out_shape=jax.ShapeDtypeStruct((N, H), jnp.float32),
    )(acc, x, W_l, b_l, W_r)


def kernel(x, edge_weight, W_l, b_l, W_r, edge_index):
    del edge_weight
    pad = jnp.zeros((N, PAD), jnp.float32).at[:, 0].set(1.0)
    xp = jnp.concatenate([x, pad], axis=1)
    src2d = edge_index[0].reshape(NW, NB, K)
    dst2d = edge_index[1].reshape(NW, NB, K)
    zrows = jnp.zeros((K, W), jnp.float32)
    acc = _sc_segment_sum(xp, src2d, dst2d, zrows)
    return _tc_dense(acc, x, W_l, b_l.reshape(1, H), W_r)


# speedup vs baseline: 10.8465x; 1.4573x over previous
import functools

import jax
import jax.numpy as jnp
from jax import lax
from jax.experimental import pallas as pl
from jax.experimental.pallas import tpu as pltpu
from jax.experimental.pallas import tpu_sc as plsc

N = 10000
E = 320000
D = 128
H = 128
PAD = 16
W = D + PAD
NC = 2
NS = 16
NW = NC * NS
EPW = E // NW
K = 80
NB = EPW // K
NA = 10240
RPT = NA // NS
CB = 25
NCH = NB // CB


def _sc_segment_sum(xp, src2d, dst2d, zrows):
    mesh = plsc.VectorSubcoreMesh(core_axis_name="c", subcore_axis_name="s")

    @functools.partial(
        pl.kernel,
        out_type=jax.ShapeDtypeStruct((NC, NA, W), jnp.float32),
        mesh=mesh,
        scratch_types=[
            pltpu.VMEM_SHARED((NA, W), jnp.float32),
            pltpu.VMEM((CB, K), jnp.int32),
            pltpu.VMEM((CB, K), jnp.int32),
            pltpu.VMEM((CB, K), jnp.int32),
            pltpu.VMEM((CB, K), jnp.int32),
            pltpu.VMEM((K, W), jnp.float32),
            pltpu.VMEM((K, W), jnp.float32),
            pltpu.SemaphoreType.DMA,
            pltpu.SemaphoreType.DMA,
            pltpu.SemaphoreType.DMA,
            pltpu.SemaphoreType.DMA,
        ],
        compiler_params=pltpu.CompilerParams(use_tc_tiling_on_sc=False),
    )
    def k(xp_hbm, src_hbm, dst_hbm, z_hbm, out_hbm,
          accum, src_a, dst_a, src_b, dst_b, rows0, rows1,
          gsem0, gsem1, isem, wsem):
        cid = lax.axis_index("c")
        sid = lax.axis_index("s")
        wid = sid * NC + cid
        srcs = (src_a, src_b)
        dsts = (dst_a, dst_b)

        pltpu.sync_copy(z_hbm, rows0)
        row0 = sid * RPT
        zh = []
        for t in range(RPT // K):
            zh.append(pltpu.async_copy(
                rows0, accum.at[pl.ds(row0 + t * K, K)], wsem))
        ih = [pltpu.async_copy(src_hbm.at[wid, pl.ds(0, CB)], src_a, isem),
              pltpu.async_copy(dst_hbm.at[wid, pl.ds(0, CB)], dst_a, isem)]
        for h in zh:
            h.wait()

        plsc.subcore_barrier()

        for c in range(NCH):
            src_v = srcs[c % 2]
            dst_v = dsts[c % 2]
            for h in ih:
                h.wait()
            if c + 1 < NCH:
                nsrc = srcs[(c + 1) % 2]
                ndst = dsts[(c + 1) % 2]
                ih = [pltpu.async_copy(
                          src_hbm.at[wid, pl.ds((c + 1) * CB, CB)], nsrc, isem),
                      pltpu.async_copy(
                          dst_hbm.at[wid, pl.ds((c + 1) * CB, CB)], ndst, isem)]

            pltpu.async_copy(xp_hbm.at[src_v.at[0]], rows0, gsem0)

            def pair(j, carry):
                i0 = 2 * j
                h1 = pltpu.async_copy(xp_hbm.at[src_v.at[i0 + 1]], rows1, gsem1)
                pltpu.make_async_copy(xp_hbm.at[src_v.at[0]], rows0,
                                      gsem0).wait()
                pltpu.sync_copy(rows0, accum.at[dst_v.at[i0]], add=True)
                pltpu.async_copy(xp_hbm.at[src_v.at[i0 + 2]], rows0, gsem0)
                h1.wait()
                pltpu.sync_copy(rows1, accum.at[dst_v.at[i0 + 1]], add=True)
                return carry

            lax.fori_loop(0, CB // 2, pair, 0)
            pltpu.make_async_copy(xp_hbm.at[src_v.at[0]], rows0, gsem0).wait()
            pltpu.sync_copy(rows0, accum.at[dst_v.at[CB - 1]], add=True)

        plsc.subcore_barrier()

        wh = [None, None]
        bufs = (rows0, rows1)
        for t in range(RPT // K):
            b = t % 2
            r = row0 + t * K
            if wh[b] is not None:
                wh[b].wait()
            pltpu.sync_copy(accum.at[pl.ds(r, K)], bufs[b])
            wh[b] = pltpu.async_copy(bufs[b], out_hbm.at[cid, pl.ds(r, K)],
                                     wsem)
        for h in wh:
            h.wait()

    return k(xp, src2d, dst2d, zrows)


def _tc_dense(acc, x, W_l, b_l, W_r):
    BN = 1000
    grid = (N // BN,)

    def body(acc_ref, x_ref, wl_ref, b_ref, wr_ref, o_ref):
        a = acc_ref[0] + acc_ref[1]
        s = a[:, :D]
        c = jnp.sum(a[:, D:], axis=1, keepdims=True)
        mean = s / jnp.maximum(c, 1.0)
        out = lax.dot_general(mean, wl_ref[...], (((1,), (1,)), ((), ())),
                              preferred_element_type=jnp.float32)
        out = out + lax.dot_general(x_ref[...], wr_ref[...],
                                    (((1,), (1,)), ((), ())),
                                    preferred_element_type=jnp.float32)
        o_ref[...] = jnp.maximum(out + b_ref[...], 0.0)

    return pl.pallas_call(
        body,
        grid=grid,
        in_specs=[
            pl.BlockSpec((NC, BN, W), lambda i: (0, i, 0)),
            pl.BlockSpec((BN, D), lambda i: (i, 0)),
            pl.BlockSpec((H, D), lambda i: (0, 0)),
            pl.BlockSpec((1, H), lambda i: (0, 0)),
            pl.BlockSpec((H, D), lambda i: (0, 0)),
        ],
        out_specs=pl.BlockSpec((BN, H), lambda i: (i, 0)),
        out_shape=jax.ShapeDtypeStruct((N, H), jnp.float32),
    )(acc, x, W_l, b_l, W_r)


def kernel(x, edge_weight, W_l, b_l, W_r, edge_index):
    del edge_weight
    pad = jnp.zeros((N, PAD), jnp.float32).at[:, 0].set(1.0)
    xp = jnp.concatenate([x, pad], axis=1)
    src2d = edge_index[0].reshape(NW, NB, K)
    dst2d = edge_index[1].reshape(NW, NB, K)
    zrows = jnp.zeros((K, W), jnp.float32)
    acc = _sc_segment_sum(xp, src2d, dst2d, zrows)
    return _tc_dense(acc, x, W_l, b_l.reshape(1, H), W_r)
